# Initial kernel scaffold; baseline (speedup 1.0000x reference)
import jax
import jax.numpy as jnp
from jax.experimental import pallas as pl


def kernel(atom_list, bond_list, atom_degree_list, bond_degree_list, atom_mask, atom_fc_W, atom_fc_b, nb_fc_W, nb_fc_b, align_W, align_b, attend_W, attend_b, gru_Wih, gru_Whh, gru_bih, gru_bhh, mol_align_W, mol_align_b, mol_attend_W, mol_attend_b, mol_gru_Wih, mol_gru_Whh, mol_gru_bih, mol_gru_bhh, dnn_W, dnn_b):
    raise NotImplementedError("write your pallas kernel here")


# speedup vs baseline: 13.6090x; 13.6090x over previous
import functools

import jax
import jax.numpy as jnp
from jax.experimental import pallas as pl


def _leaky(x):
    return jnp.where(x >= 0, x, 0.01 * x)


def _elu(x):
    return jnp.where(x > 0, x, jnp.exp(jnp.minimum(x, 0.0)) - 1.0)


def _body(M, L, D, F,
          atom_ref, bond_ref, aidx_ref, bidx_ref, mask_ref,
          atom_fc_W_ref, atom_fc_b_ref, Wnba_ref, Wnbb_ref, nbfb_ref,
          align_wa_ref, align_wn_ref, align_b_ref,
          attend_W_ref, attend_b_ref,
          WihT_ref, WhhT_ref, bih_ref, bhh_ref,
          mol_wa_ref, mol_wn_ref, mol_b_ref,
          mol_attW_ref, mol_attb_ref,
          mWihT_ref, mWhhT_ref, mbih_ref, mbhh_ref,
          dnnW_ref, dnnb_ref,
          out_ref):
    f32 = jnp.float32
    ML = M * L
    DL = D * L

    atom = atom_ref[...].reshape(ML, atom_ref.shape[-1])
    bond = bond_ref[...].reshape(ML, bond_ref.shape[-1])

    dot = functools.partial(jnp.dot, preferred_element_type=f32)

    af = _leaky(dot(atom, atom_fc_W_ref[...]) + atom_fc_b_ref[...])
    pa = dot(atom, Wnba_ref[...])
    pb = dot(bond, Wnbb_ref[...])

    iota = jax.lax.broadcasted_iota(jnp.int32, (DL, L), 1)

    Oa = []
    nbf = []
    amask = []
    smask = []
    for m in range(M):
        aidx_m = aidx_ref[m]
        bidx_m = bidx_ref[m]
        oa = (aidx_m == iota).astype(f32)
        ob = (bidx_m == iota).astype(f32)
        ga = dot(oa, pa[m * L:(m + 1) * L])
        gb = dot(ob, pb[m * L:(m + 1) * L])
        nbf.append(_leaky(ga + gb + nbfb_ref[...]))
        Oa.append(oa)
        is_pad = aidx_m == (L - 1)
        amask.append(jnp.where(is_pad, 0.0, 1.0).astype(f32))
        smask.append(jnp.where(is_pad, -9e8, 0.0).astype(f32))

    h = af
    atom_feat = af
    for r in range(3):
        wa = align_wa_ref[r]
        wn = align_wn_ref[r]
        ab = align_b_ref[r]
        sa = jnp.sum(atom_feat * wa, axis=-1, keepdims=True)
        if r > 0:
            activated = jnp.maximum(h, 0.0)
            at = dot(activated, attend_W_ref[r]) + attend_b_ref[r]
            snv = jnp.sum(activated * wn, axis=-1, keepdims=True)
        ctx_rows = []
        for m in range(M):
            sl = slice(m * L, (m + 1) * L)
            if r == 0:
                nbf_m = nbf[m]
                sn_col = jnp.sum(nbf_m * wn, axis=-1, keepdims=True)
                nft_m = dot(nbf_m, attend_W_ref[r]) + attend_b_ref[r]
            else:
                sn_col = dot(Oa[m], snv[sl])
            sa_m = sa[sl]
            score = [
                _leaky(sa_m + sn_col[d * L:(d + 1) * L] + ab)
                + smask[m][d * L:(d + 1) * L]
                for d in range(D)
            ]
            mx = score[0]
            for d in range(1, D):
                mx = jnp.maximum(mx, score[d])
            ex = [jnp.exp(score[d] - mx) for d in range(D)]
            se = ex[0]
            for d in range(1, D):
                se = se + ex[d]
            inv = 1.0 / se
            if r == 0:
                ctx_m = jnp.zeros((L, F), f32)
                for d in range(D):
                    aw_d = ex[d] * inv * amask[m][d * L:(d + 1) * L]
                    ctx_m = ctx_m + aw_d * nft_m[d * L:(d + 1) * L]
            else:
                Wg = jnp.zeros((L, L), f32)
                for d in range(D):
                    aw_d = ex[d] * inv * amask[m][d * L:(d + 1) * L]
                    Wg = Wg + aw_d * Oa[m][d * L:(d + 1) * L]
                ctx_m = dot(Wg, at[sl])
            ctx_rows.append(_elu(ctx_m))
        ctx = jnp.concatenate(ctx_rows, axis=0)

        gi = dot(ctx, WihT_ref[r]) + bih_ref[r]
        gh = dot(h, WhhT_ref[r]) + bhh_ref[r]
        rr = jax.nn.sigmoid(gi[:, 0:F] + gh[:, 0:F])
        zz = jax.nn.sigmoid(gi[:, F:2 * F] + gh[:, F:2 * F])
        nn = jnp.tanh(gi[:, 2 * F:3 * F] + rr * gh[:, 2 * F:3 * F])
        h = (1.0 - zz) * nn + zz * h
        atom_feat = h

    activated = jnp.maximum(h, 0.0)

    aft = dot(activated, mol_attW_ref[...]) + mol_attb_ref[...]
    s_act = jnp.sum(activated * mol_wn_ref[...], axis=-1, keepdims=True)

    mf_rows = []
    for m in range(M):
        sl = slice(m * L, (m + 1) * L)
        mf_rows.append(jnp.sum(activated[sl] * mask_ref[m], axis=0, keepdims=True))
    mol_f = jnp.concatenate(mf_rows, axis=0)

    for _t in range(2):
        act_mol = jnp.maximum(mol_f, 0.0)
        s_mol = jnp.sum(act_mol * mol_wa_ref[...], axis=-1, keepdims=True)
        ctx_rows = []
        for m in range(M):
            sl = slice(m * L, (m + 1) * L)
            mask_m = mask_ref[m]
            msk = jnp.where(mask_m == 0, -9e8, 0.0).astype(f32)
            score = _leaky(s_mol[m:m + 1] + s_act[sl] + mol_b_ref[...]) + msk
            mx = jnp.max(score, axis=0, keepdims=True)
            ex = jnp.exp(score - mx)
            se = jnp.sum(ex, axis=0, keepdims=True)
            maw = ex / se * mask_m
            ctx_rows.append(jnp.sum(maw * aft[sl], axis=0, keepdims=True))
        ctx = _elu(jnp.concatenate(ctx_rows, axis=0))

        gi = dot(ctx, mWihT_ref[...]) + mbih_ref[...]
        gh = dot(mol_f, mWhhT_ref[...]) + mbhh_ref[...]
        rr = jax.nn.sigmoid(gi[:, 0:F] + gh[:, 0:F])
        zz = jax.nn.sigmoid(gi[:, F:2 * F] + gh[:, F:2 * F])
        nn = jnp.tanh(gi[:, 2 * F:3 * F] + rr * gh[:, 2 * F:3 * F])
        mol_f = (1.0 - zz) * nn + zz * mol_f

    out_ref[...] = dot(mol_f, dnnW_ref[...]) + dnnb_ref[...]


def kernel(atom_list, bond_list, atom_degree_list, bond_degree_list, atom_mask,
           atom_fc_W, atom_fc_b, nb_fc_W, nb_fc_b, align_W, align_b,
           attend_W, attend_b, gru_Wih, gru_Whh, gru_bih, gru_bhh,
           mol_align_W, mol_align_b, mol_attend_W, mol_attend_b,
           mol_gru_Wih, mol_gru_Whh, mol_gru_bih, mol_gru_bhh, dnn_W, dnn_b):
    f32 = jnp.float32
    B, L, IN_ATOM = atom_list.shape
    IN_BOND = bond_list.shape[-1]
    D = atom_degree_list.shape[-1]
    F = atom_fc_W.shape[1]
    R = align_W.shape[0]
    M = 8
    NP = 128

    aidx = atom_degree_list.astype(jnp.int32).transpose(0, 2, 1).reshape(B, D * L, 1)
    bidx = bond_degree_list.astype(jnp.int32).transpose(0, 2, 1).reshape(B, D * L, 1)
    mask3 = atom_mask.astype(f32).reshape(B, L, 1)

    afb = atom_fc_b.reshape(1, F)
    Wnba = nb_fc_W[:IN_ATOM]
    Wnbb = nb_fc_W[IN_ATOM:]
    nbfb = nb_fc_b.reshape(1, F)
    align_wa = align_W[:, :F, 0].reshape(R, 1, F)
    align_wn = align_W[:, F:, 0].reshape(R, 1, F)
    align_b3 = align_b.reshape(R, 1, 1)
    attend_b3 = attend_b.reshape(R, 1, F)
    WihT = jnp.transpose(gru_Wih, (0, 2, 1))
    WhhT = jnp.transpose(gru_Whh, (0, 2, 1))
    bih3 = gru_bih.reshape(R, 1, 3 * F)
    bhh3 = gru_bhh.reshape(R, 1, 3 * F)
    mol_wa = mol_align_W[:F, 0].reshape(1, F)
    mol_wn = mol_align_W[F:, 0].reshape(1, F)
    mol_b2 = mol_align_b.reshape(1, 1)
    molab = mol_attend_b.reshape(1, F)
    mWihT = mol_gru_Wih.T
    mWhhT = mol_gru_Whh.T
    mbih = mol_gru_bih.reshape(1, 3 * F)
    mbhh = mol_gru_bhh.reshape(1, 3 * F)
    dnnW_p = jnp.zeros((F, NP), f32).at[:, :dnn_W.shape[1]].set(dnn_W)
    dnnb_p = jnp.zeros((1, NP), f32).at[0, :dnn_b.shape[0]].set(dnn_b)

    def fixed(a):
        nd = a.ndim
        return pl.BlockSpec(a.shape, lambda i, _nd=nd: (0,) * _nd)

    weights = [atom_fc_W, afb, Wnba, Wnbb, nbfb,
               align_wa, align_wn, align_b3, attend_W, attend_b3,
               WihT, WhhT, bih3, bhh3,
               mol_wa, mol_wn, mol_b2, mol_attend_W, molab,
               mWihT, mWhhT, mbih, mbhh, dnnW_p, dnnb_p]

    out = pl.pallas_call(
        functools.partial(_body, M, L, D, F),
        grid=(B // M,),
        in_specs=[
            pl.BlockSpec((M, L, IN_ATOM), lambda i: (i, 0, 0)),
            pl.BlockSpec((M, L, IN_BOND), lambda i: (i, 0, 0)),
            pl.BlockSpec((M, D * L, 1), lambda i: (i, 0, 0)),
            pl.BlockSpec((M, D * L, 1), lambda i: (i, 0, 0)),
            pl.BlockSpec((M, L, 1), lambda i: (i, 0, 0)),
        ] + [fixed(w) for w in weights],
        out_specs=pl.BlockSpec((M, NP), lambda i: (i, 0)),
        out_shape=jax.ShapeDtypeStruct((B, NP), f32),
    )(atom_list, bond_list, aidx, bidx, mask3, *weights)

    return out[:, :dnn_W.shape[1]]


# speedup vs baseline: 21.4812x; 1.5785x over previous
import functools

import jax
import jax.numpy as jnp
from jax.experimental import pallas as pl


def _leaky(x):
    return jnp.where(x >= 0, x, 0.01 * x)


def _elu(x):
    return jnp.where(x > 0, x, jnp.exp(jnp.minimum(x, 0.0)) - 1.0)


def _body(M, L, D, F,
          atomT_ref, bondT_ref, aidx_ref, bidx_ref, maskT_ref, mcol_ref,
          atom_fcT_ref, atom_fcbT_ref, WnbaT_ref, WnbbT_ref, nbfbT_ref,
          align_wa_ref, align_wn_ref, align_b_ref,
          attend_WT_ref, attend_bT_ref,
          Wih_ref, Whh_ref, bihT_ref, bhhT_ref,
          mol_wa_ref, mol_wn_ref, mol_b_ref,
          mol_attWT_ref, mol_attbT_ref,
          mWih_ref, mWhh_ref, mbihT_ref, mbhhT_ref,
          dnnWT_ref, dnnbT_ref,
          out_ref):
    f32 = jnp.float32
    ML = M * L
    DL = D * L

    dot = functools.partial(jnp.dot, preferred_element_type=f32)

    atomT = jnp.concatenate([atomT_ref[m] for m in range(M)], axis=1)
    bondT = jnp.concatenate([bondT_ref[m] for m in range(M)], axis=1)

    afT = _leaky(dot(atom_fcT_ref[...], atomT) + atom_fcbT_ref[...])
    paT = dot(WnbaT_ref[...], atomT)
    pbT = dot(WnbbT_ref[...], bondT)

    iota = jax.lax.broadcasted_iota(jnp.int32, (L, DL), 0)

    OaT = []
    nbfT = []
    amask = []
    smask = []
    for m in range(M):
        aidx_m = aidx_ref[m]
        bidx_m = bidx_ref[m]
        oaT = (iota == aidx_m).astype(f32)
        obT = (iota == bidx_m).astype(f32)
        sl = slice(m * L, (m + 1) * L)
        gaT = dot(paT[:, sl], oaT)
        gbT = dot(pbT[:, sl], obT)
        nbfT.append(_leaky(gaT + gbT + nbfbT_ref[...]))
        OaT.append(oaT)
        is_pad = aidx_m == (L - 1)
        amask.append(jnp.where(is_pad, 0.0, 1.0).astype(f32))
        smask.append(jnp.where(is_pad, -9e8, 0.0).astype(f32))

    hT = afT
    atom_featT = afT
    for r in range(3):
        wa = align_wa_ref[r]
        wn = align_wn_ref[r]
        ab = align_b_ref[r]
        saT = dot(wa, atom_featT)
        if r > 0:
            activT = jnp.maximum(hT, 0.0)
            atT = dot(attend_WT_ref[r], activT) + attend_bT_ref[r]
            snvT = dot(wn, activT)
        ctx_cols = []
        for m in range(M):
            sl = slice(m * L, (m + 1) * L)
            if r == 0:
                nbf_m = nbfT[m]
                sn_row = dot(wn, nbf_m)
                nft_m = dot(attend_WT_ref[r], nbf_m) + attend_bT_ref[r]
            else:
                sn_row = dot(snvT[:, sl], OaT[m])
            sa_m = saT[:, sl]
            score = [
                _leaky(sa_m + sn_row[:, d * L:(d + 1) * L] + ab)
                + smask[m][:, d * L:(d + 1) * L]
                for d in range(D)
            ]
            mx01 = jnp.maximum(score[0], score[1])
            mx23 = jnp.maximum(score[2], score[3])
            mx45 = jnp.maximum(score[4], score[5])
            mx = jnp.maximum(jnp.maximum(mx01, mx23), mx45)
            ex = [jnp.exp(score[d] - mx) for d in range(D)]
            se = (ex[0] + ex[1]) + (ex[2] + ex[3]) + (ex[4] + ex[5])
            inv = 1.0 / se
            aw = [ex[d] * inv * amask[m][:, d * L:(d + 1) * L] for d in range(D)]
            if r == 0:
                ctx_m = aw[0] * nft_m[:, 0 * L:1 * L]
                for d in range(1, D):
                    ctx_m = ctx_m + aw[d] * nft_m[:, d * L:(d + 1) * L]
            else:
                WgT = aw[0] * OaT[m][:, 0 * L:1 * L]
                for d in range(1, D):
                    WgT = WgT + aw[d] * OaT[m][:, d * L:(d + 1) * L]
                ctx_m = dot(atT[:, sl], WgT)
            ctx_cols.append(_elu(ctx_m))
        ctxT = jnp.concatenate(ctx_cols, axis=1)

        giT = dot(Wih_ref[r], ctxT) + bihT_ref[r]
        ghT = dot(Whh_ref[r], hT) + bhhT_ref[r]
        rr = jax.nn.sigmoid(giT[0:F] + ghT[0:F])
        zz = jax.nn.sigmoid(giT[F:2 * F] + ghT[F:2 * F])
        nn = jnp.tanh(giT[2 * F:3 * F] + rr * ghT[2 * F:3 * F])
        hT = (1.0 - zz) * nn + zz * hT
        atom_featT = hT

    activT = jnp.maximum(hT, 0.0)

    mol_fT = dot(activT, mcol_ref[0])
    aftT = dot(mol_attWT_ref[...], activT) + mol_attbT_ref[...]
    s_actT = dot(mol_wn_ref[...], activT)

    for _t in range(2):
        act_molT = jnp.maximum(mol_fT, 0.0)
        s_molT = dot(mol_wa_ref[...], act_molT)
        ctx_cols = []
        for m in range(M):
            sl = slice(m * L, (m + 1) * L)
            mask_m = maskT_ref[m]
            msk = jnp.where(mask_m == 0, -9e8, 0.0).astype(f32)
            score = _leaky(s_molT[:, m:m + 1] + s_actT[:, sl] + mol_b_ref[...]) + msk
            mx = jnp.max(score, axis=1, keepdims=True)
            ex = jnp.exp(score - mx)
            se = jnp.sum(ex, axis=1, keepdims=True)
            maw = ex / se * mask_m
            ctx_cols.append(jnp.sum(aftT[:, sl] * maw, axis=1, keepdims=True))
        ctxT = _elu(jnp.concatenate(ctx_cols, axis=1))

        giT = dot(mWih_ref[...], ctxT) + mbihT_ref[...]
        ghT = dot(mWhh_ref[...], mol_fT) + mbhhT_ref[...]
        rr = jax.nn.sigmoid(giT[0:F] + ghT[0:F])
        zz = jax.nn.sigmoid(giT[F:2 * F] + ghT[F:2 * F])
        nn = jnp.tanh(giT[2 * F:3 * F] + rr * ghT[2 * F:3 * F])
        mol_fT = (1.0 - zz) * nn + zz * mol_fT

    out_ref[...] = (dot(dnnWT_ref[...], mol_fT) + dnnbT_ref[...])[None]


def kernel(atom_list, bond_list, atom_degree_list, bond_degree_list, atom_mask,
           atom_fc_W, atom_fc_b, nb_fc_W, nb_fc_b, align_W, align_b,
           attend_W, attend_b, gru_Wih, gru_Whh, gru_bih, gru_bhh,
           mol_align_W, mol_align_b, mol_attend_W, mol_attend_b,
           mol_gru_Wih, mol_gru_Whh, mol_gru_bih, mol_gru_bhh, dnn_W, dnn_b):
    f32 = jnp.float32
    B, L, IN_ATOM = atom_list.shape
    IN_BOND = bond_list.shape[-1]
    D = atom_degree_list.shape[-1]
    F = atom_fc_W.shape[1]
    R = align_W.shape[0]
    M = 8
    NO = dnn_W.shape[1]
    NP = 128

    atomT = atom_list.transpose(0, 2, 1)
    bondT = bond_list.transpose(0, 2, 1)
    aidx = atom_degree_list.astype(jnp.int32).transpose(0, 2, 1).reshape(B, 1, D * L)
    bidx = bond_degree_list.astype(jnp.int32).transpose(0, 2, 1).reshape(B, 1, D * L)
    maskT = atom_mask.astype(f32).reshape(B, 1, L)
    eye = jnp.eye(M, dtype=f32)
    mcol = (atom_mask.astype(f32).reshape(B // M, M, L)[:, :, :, None]
            * eye[:, None, :]).reshape(B // M, M * L, M)

    atom_fcT = atom_fc_W.T
    atom_fcbT = atom_fc_b.reshape(F, 1)
    WnbaT = nb_fc_W[:IN_ATOM].T
    WnbbT = nb_fc_W[IN_ATOM:].T
    nbfbT = nb_fc_b.reshape(F, 1)
    align_wa = align_W[:, :F, 0].reshape(R, 1, F)
    align_wn = align_W[:, F:, 0].reshape(R, 1, F)
    align_b3 = align_b.reshape(R, 1, 1)
    attend_WT = jnp.transpose(attend_W, (0, 2, 1))
    attend_bT = attend_b.reshape(R, F, 1)
    bihT = gru_bih.reshape(R, 3 * F, 1)
    bhhT = gru_bhh.reshape(R, 3 * F, 1)
    mol_wa = mol_align_W[:F, 0].reshape(1, F)
    mol_wn = mol_align_W[F:, 0].reshape(1, F)
    mol_b2 = mol_align_b.reshape(1, 1)
    mol_attWT = mol_attend_W.T
    mol_attbT = mol_attend_b.reshape(F, 1)
    mbihT = mol_gru_bih.reshape(3 * F, 1)
    mbhhT = mol_gru_bhh.reshape(3 * F, 1)
    dnnWT_p = jnp.zeros((NP, F), f32).at[:NO, :].set(dnn_W.T)
    dnnbT_p = jnp.zeros((NP, 1), f32).at[:NO, 0].set(dnn_b)

    def fixed(a):
        nd = a.ndim
        return pl.BlockSpec(a.shape, lambda i, _nd=nd: (0,) * _nd)

    weights = [atom_fcT, atom_fcbT, WnbaT, WnbbT, nbfbT,
               align_wa, align_wn, align_b3, attend_WT, attend_bT,
               gru_Wih, gru_Whh, bihT, bhhT,
               mol_wa, mol_wn, mol_b2, mol_attWT, mol_attbT,
               mol_gru_Wih, mol_gru_Whh, mbihT, mbhhT, dnnWT_p, dnnbT_p]

    out = pl.pallas_call(
        functools.partial(_body, M, L, D, F),
        grid=(B // M,),
        in_specs=[
            pl.BlockSpec((M, IN_ATOM, L), lambda i: (i, 0, 0)),
            pl.BlockSpec((M, IN_BOND, L), lambda i: (i, 0, 0)),
            pl.BlockSpec((M, 1, D * L), lambda i: (i, 0, 0)),
            pl.BlockSpec((M, 1, D * L), lambda i: (i, 0, 0)),
            pl.BlockSpec((M, 1, L), lambda i: (i, 0, 0)),
            pl.BlockSpec((1, M * L, M), lambda i: (i, 0, 0)),
        ] + [fixed(w) for w in weights],
        out_specs=pl.BlockSpec((1, NP, M), lambda i: (i, 0, 0)),
        out_shape=jax.ShapeDtypeStruct((B // M, NP, M), f32),
    )(atomT, bondT, aidx, bidx, maskT, mcol, *weights)

    return out.transpose(0, 2, 1).reshape(B, NP)[:, :NO]


# speedup vs baseline: 26.8176x; 1.2484x over previous
import functools

import jax
import jax.numpy as jnp
from jax.experimental import pallas as pl


def _leaky(x):
    return jnp.where(x >= 0, x, 0.01 * x)


def _elu(x):
    return jnp.where(x > 0, x, jnp.exp(jnp.minimum(x, 0.0)) - 1.0)


def _body(M, L, D, F,
          atomT_ref, bondT_ref, aidx_ref, bidx_ref, maskT_ref, mcol_ref,
          atom_fcT_ref, atom_fcbT_ref, WnbaT_ref, WnbbT_ref, nbfbT_ref,
          align_wa_ref, align_wn_ref, align_b_ref,
          attend_WT_ref, attend_bT_ref,
          Wih_ref, Whh_ref, bihT_ref, bhhT_ref,
          mol_wa_ref, mol_wn_ref, mol_b_ref,
          mol_attWT_ref, mol_attbT_ref,
          mWih_ref, mWhh_ref, mbihT_ref, mbhhT_ref,
          dnnWT_ref, dnnbT_ref,
          out_ref):
    f32 = jnp.float32
    ML = M * L
    DL = D * L

    dot = functools.partial(jnp.dot, preferred_element_type=f32)

    atomT = jnp.concatenate([atomT_ref[m] for m in range(M)], axis=1)
    bondT = jnp.concatenate([bondT_ref[m] for m in range(M)], axis=1)

    afT = _leaky(dot(atom_fcT_ref[...], atomT) + atom_fcbT_ref[...])
    paT = dot(WnbaT_ref[...], atomT)
    pbT = dot(WnbbT_ref[...], bondT)

    iota = jax.lax.broadcasted_iota(jnp.int32, (L, DL), 0)

    OaT = []
    nbfT = []
    amask = []
    smask = []
    for m in range(M):
        aidx_m = aidx_ref[m]
        bidx_m = bidx_ref[m]
        oaT = (iota == aidx_m).astype(f32)
        obT = (iota == bidx_m).astype(f32)
        sl = slice(m * L, (m + 1) * L)
        gaT = dot(paT[:, sl], oaT)
        gbT = dot(pbT[:, sl], obT)
        nbfT.append(_leaky(gaT + gbT + nbfbT_ref[...]))
        OaT.append(oaT)
        is_pad = aidx_m == (L - 1)
        amask.append(jnp.where(is_pad, 0.0, 1.0).astype(f32))
        smask.append(jnp.where(is_pad, -9e8, 0.0).astype(f32))

    hT = afT
    atom_featT = afT
    for r in range(3):
        wa = align_wa_ref[r]
        wn = align_wn_ref[r]
        ab = align_b_ref[r]
        saT = dot(wa, atom_featT)
        if r > 0:
            activT = jnp.maximum(hT, 0.0)
            atT = dot(attend_WT_ref[r], activT) + attend_bT_ref[r]
            snvT = dot(wn, activT)
        ctx_cols = []
        for m in range(M):
            sl = slice(m * L, (m + 1) * L)
            if r == 0:
                nbf_m = nbfT[m]
                sn_row = dot(wn, nbf_m)
                nft_m = dot(attend_WT_ref[r], nbf_m) + attend_bT_ref[r]
            else:
                sn_row = dot(snvT[:, sl], OaT[m])
            sa_m = saT[:, sl]
            score = [
                _leaky(sa_m + sn_row[:, d * L:(d + 1) * L] + ab)
                + smask[m][:, d * L:(d + 1) * L]
                for d in range(D)
            ]
            mx01 = jnp.maximum(score[0], score[1])
            mx23 = jnp.maximum(score[2], score[3])
            mx45 = jnp.maximum(score[4], score[5])
            mx = jnp.maximum(jnp.maximum(mx01, mx23), mx45)
            ex = [jnp.exp(score[d] - mx) for d in range(D)]
            se = (ex[0] + ex[1]) + (ex[2] + ex[3]) + (ex[4] + ex[5])
            inv = 1.0 / se
            aw = [ex[d] * inv * amask[m][:, d * L:(d + 1) * L] for d in range(D)]
            if r == 0:
                ctx_m = aw[0] * nft_m[:, 0 * L:1 * L]
                for d in range(1, D):
                    ctx_m = ctx_m + aw[d] * nft_m[:, d * L:(d + 1) * L]
            else:
                WgT = aw[0] * OaT[m][:, 0 * L:1 * L]
                for d in range(1, D):
                    WgT = WgT + aw[d] * OaT[m][:, d * L:(d + 1) * L]
                ctx_m = dot(atT[:, sl], WgT)
            ctx_cols.append(_elu(ctx_m))
        ctxT = jnp.concatenate(ctx_cols, axis=1)

        giT = dot(Wih_ref[r], ctxT) + bihT_ref[r]
        ghT = dot(Whh_ref[r], hT) + bhhT_ref[r]
        rr = jax.nn.sigmoid(giT[0:F] + ghT[0:F])
        zz = jax.nn.sigmoid(giT[F:2 * F] + ghT[F:2 * F])
        nn = jnp.tanh(giT[2 * F:3 * F] + rr * ghT[2 * F:3 * F])
        hT = (1.0 - zz) * nn + zz * hT
        atom_featT = hT

    activT = jnp.maximum(hT, 0.0)

    mol_fT = dot(activT, mcol_ref[0])
    aftT = dot(mol_attWT_ref[...], activT) + mol_attbT_ref[...]
    s_actT = dot(mol_wn_ref[...], activT)

    for _t in range(2):
        act_molT = jnp.maximum(mol_fT, 0.0)
        s_molT = dot(mol_wa_ref[...], act_molT)
        ctx_cols = []
        for m in range(M):
            sl = slice(m * L, (m + 1) * L)
            mask_m = maskT_ref[m]
            msk = jnp.where(mask_m == 0, -9e8, 0.0).astype(f32)
            score = _leaky(s_molT[:, m:m + 1] + s_actT[:, sl] + mol_b_ref[...]) + msk
            mx = jnp.max(score, axis=1, keepdims=True)
            ex = jnp.exp(score - mx)
            se = jnp.sum(ex, axis=1, keepdims=True)
            maw = ex / se * mask_m
            ctx_cols.append(jnp.sum(aftT[:, sl] * maw, axis=1, keepdims=True))
        ctxT = _elu(jnp.concatenate(ctx_cols, axis=1))

        giT = dot(mWih_ref[...], ctxT) + mbihT_ref[...]
        ghT = dot(mWhh_ref[...], mol_fT) + mbhhT_ref[...]
        rr = jax.nn.sigmoid(giT[0:F] + ghT[0:F])
        zz = jax.nn.sigmoid(giT[F:2 * F] + ghT[F:2 * F])
        nn = jnp.tanh(giT[2 * F:3 * F] + rr * ghT[2 * F:3 * F])
        mol_fT = (1.0 - zz) * nn + zz * mol_fT

    out_ref[...] = (dot(dnnWT_ref[...], mol_fT) + dnnbT_ref[...])[None]


def kernel(atom_list, bond_list, atom_degree_list, bond_degree_list, atom_mask,
           atom_fc_W, atom_fc_b, nb_fc_W, nb_fc_b, align_W, align_b,
           attend_W, attend_b, gru_Wih, gru_Whh, gru_bih, gru_bhh,
           mol_align_W, mol_align_b, mol_attend_W, mol_attend_b,
           mol_gru_Wih, mol_gru_Whh, mol_gru_bih, mol_gru_bhh, dnn_W, dnn_b):
    f32 = jnp.float32
    B, L, IN_ATOM = atom_list.shape
    IN_BOND = bond_list.shape[-1]
    D = atom_degree_list.shape[-1]
    F = atom_fc_W.shape[1]
    R = align_W.shape[0]
    M = 16
    NO = dnn_W.shape[1]
    NP = 128

    atomT = atom_list.transpose(0, 2, 1)
    bondT = bond_list.transpose(0, 2, 1)
    aidx = atom_degree_list.astype(jnp.int32).transpose(0, 2, 1).reshape(B, 1, D * L)
    bidx = bond_degree_list.astype(jnp.int32).transpose(0, 2, 1).reshape(B, 1, D * L)
    maskT = atom_mask.astype(f32).reshape(B, 1, L)
    eye = jnp.eye(M, dtype=f32)
    mcol = (atom_mask.astype(f32).reshape(B // M, M, L)[:, :, :, None]
            * eye[:, None, :]).reshape(B // M, M * L, M)

    atom_fcT = atom_fc_W.T
    atom_fcbT = atom_fc_b.reshape(F, 1)
    WnbaT = nb_fc_W[:IN_ATOM].T
    WnbbT = nb_fc_W[IN_ATOM:].T
    nbfbT = nb_fc_b.reshape(F, 1)
    align_wa = align_W[:, :F, 0].reshape(R, 1, F)
    align_wn = align_W[:, F:, 0].reshape(R, 1, F)
    align_b3 = align_b.reshape(R, 1, 1)
    attend_WT = jnp.transpose(attend_W, (0, 2, 1))
    attend_bT = attend_b.reshape(R, F, 1)
    bihT = gru_bih.reshape(R, 3 * F, 1)
    bhhT = gru_bhh.reshape(R, 3 * F, 1)
    mol_wa = mol_align_W[:F, 0].reshape(1, F)
    mol_wn = mol_align_W[F:, 0].reshape(1, F)
    mol_b2 = mol_align_b.reshape(1, 1)
    mol_attWT = mol_attend_W.T
    mol_attbT = mol_attend_b.reshape(F, 1)
    mbihT = mol_gru_bih.reshape(3 * F, 1)
    mbhhT = mol_gru_bhh.reshape(3 * F, 1)
    dnnWT_p = jnp.zeros((NP, F), f32).at[:NO, :].set(dnn_W.T)
    dnnbT_p = jnp.zeros((NP, 1), f32).at[:NO, 0].set(dnn_b)

    def fixed(a):
        nd = a.ndim
        return pl.BlockSpec(a.shape, lambda i, _nd=nd: (0,) * _nd)

    weights = [atom_fcT, atom_fcbT, WnbaT, WnbbT, nbfbT,
               align_wa, align_wn, align_b3, attend_WT, attend_bT,
               gru_Wih, gru_Whh, bihT, bhhT,
               mol_wa, mol_wn, mol_b2, mol_attWT, mol_attbT,
               mol_gru_Wih, mol_gru_Whh, mbihT, mbhhT, dnnWT_p, dnnbT_p]

    out = pl.pallas_call(
        functools.partial(_body, M, L, D, F),
        grid=(B // M,),
        in_specs=[
            pl.BlockSpec((M, IN_ATOM, L), lambda i: (i, 0, 0)),
            pl.BlockSpec((M, IN_BOND, L), lambda i: (i, 0, 0)),
            pl.BlockSpec((M, 1, D * L), lambda i: (i, 0, 0)),
            pl.BlockSpec((M, 1, D * L), lambda i: (i, 0, 0)),
            pl.BlockSpec((M, 1, L), lambda i: (i, 0, 0)),
            pl.BlockSpec((1, M * L, M), lambda i: (i, 0, 0)),
        ] + [fixed(w) for w in weights],
        out_specs=pl.BlockSpec((1, NP, M), lambda i: (i, 0, 0)),
        out_shape=jax.ShapeDtypeStruct((B // M, NP, M), f32),
    )(atomT, bondT, aidx, bidx, maskT, mcol, *weights)

    return out.transpose(0, 2, 1).reshape(B, NP)[:, :NO]


# speedup vs baseline: 28.8899x; 1.0773x over previous
import functools

import jax
import jax.numpy as jnp
from jax.experimental import pallas as pl


def _leaky(x):
    return jnp.where(x >= 0, x, 0.01 * x)


def _elu(x):
    return jnp.where(x > 0, x, jnp.exp(jnp.minimum(x, 0.0)) - 1.0)


def _body(M, L, D, F,
          atomT_ref, bondT_ref, aidx_ref, bidx_ref, maskT_ref, mcol_ref,
          atom_fcT_ref, atom_fcbT_ref, WnbaT_ref, WnbbT_ref, nbfbT_ref,
          align_wa_ref, align_wn_ref, align_b_ref,
          attend_WT_ref, attend_bT_ref,
          Wih_ref, Whh_ref, bihT_ref, bhhT_ref,
          mol_wa_ref, mol_wn_ref, mol_b_ref,
          mol_attWT_ref, mol_attbT_ref,
          mWih_ref, mWhh_ref, mbihT_ref, mbhhT_ref,
          dnnWT_ref, dnnbT_ref,
          out_ref):
    f32 = jnp.float32
    ML = M * L
    DL = D * L

    dot = functools.partial(jnp.dot, preferred_element_type=f32)

    atomT = jnp.concatenate([atomT_ref[m] for m in range(M)], axis=1)
    bondT = jnp.concatenate([bondT_ref[m] for m in range(M)], axis=1)

    afT = _leaky(dot(atom_fcT_ref[...], atomT) + atom_fcbT_ref[...])
    paT = dot(WnbaT_ref[...], atomT)
    pbT = dot(WnbbT_ref[...], bondT)

    iota = jax.lax.broadcasted_iota(jnp.int32, (L, DL), 0)

    OaT = []
    nbfT = []
    amask = []
    smask = []
    for m in range(M):
        aidx_m = aidx_ref[m]
        bidx_m = bidx_ref[m]
        oaT = (iota == aidx_m).astype(f32)
        obT = (iota == bidx_m).astype(f32)
        sl = slice(m * L, (m + 1) * L)
        gaT = dot(paT[:, sl], oaT)
        gbT = dot(pbT[:, sl], obT)
        nbfT.append(_leaky(gaT + gbT + nbfbT_ref[...]))
        OaT.append(oaT)
        is_pad = aidx_m == (L - 1)
        amask.append(jnp.where(is_pad, 0.0, 1.0).astype(f32))
        smask.append(jnp.where(is_pad, -9e8, 0.0).astype(f32))

    hT = afT
    atom_featT = afT
    for r in range(3):
        wa = align_wa_ref[r]
        wn = align_wn_ref[r]
        ab = align_b_ref[r]
        saT = dot(wa, atom_featT)
        if r > 0:
            activT = jnp.maximum(hT, 0.0)
            atT = dot(attend_WT_ref[r], activT) + attend_bT_ref[r]
            snvT = dot(wn, activT)
        ctx_cols = []
        for m in range(M):
            sl = slice(m * L, (m + 1) * L)
            if r == 0:
                nbf_m = nbfT[m]
                sn_row = dot(wn, nbf_m)
                nft_m = dot(attend_WT_ref[r], nbf_m) + attend_bT_ref[r]
            else:
                sn_row = dot(snvT[:, sl], OaT[m])
            sa_m = saT[:, sl]
            score = [
                _leaky(sa_m + sn_row[:, d * L:(d + 1) * L] + ab)
                + smask[m][:, d * L:(d + 1) * L]
                for d in range(D)
            ]
            mx01 = jnp.maximum(score[0], score[1])
            mx23 = jnp.maximum(score[2], score[3])
            mx45 = jnp.maximum(score[4], score[5])
            mx = jnp.maximum(jnp.maximum(mx01, mx23), mx45)
            ex = [jnp.exp(score[d] - mx) for d in range(D)]
            se = (ex[0] + ex[1]) + (ex[2] + ex[3]) + (ex[4] + ex[5])
            inv = 1.0 / se
            aw = [ex[d] * inv * amask[m][:, d * L:(d + 1) * L] for d in range(D)]
            if r == 0:
                ctx_m = aw[0] * nft_m[:, 0 * L:1 * L]
                for d in range(1, D):
                    ctx_m = ctx_m + aw[d] * nft_m[:, d * L:(d + 1) * L]
            else:
                WgT = aw[0] * OaT[m][:, 0 * L:1 * L]
                for d in range(1, D):
                    WgT = WgT + aw[d] * OaT[m][:, d * L:(d + 1) * L]
                ctx_m = dot(atT[:, sl], WgT)
            ctx_cols.append(ctx_m)
        ctxT = _elu(jnp.concatenate(ctx_cols, axis=1))

        giT = dot(Wih_ref[r], ctxT) + bihT_ref[r]
        ghT = dot(Whh_ref[r], hT) + bhhT_ref[r]
        rr = jax.nn.sigmoid(giT[0:F] + ghT[0:F])
        zz = jax.nn.sigmoid(giT[F:2 * F] + ghT[F:2 * F])
        nn = jnp.tanh(giT[2 * F:3 * F] + rr * ghT[2 * F:3 * F])
        hT = (1.0 - zz) * nn + zz * hT
        atom_featT = hT

    activT = jnp.maximum(hT, 0.0)

    mol_fT = dot(activT, mcol_ref[0])
    aftT = dot(mol_attWT_ref[...], activT) + mol_attbT_ref[...]
    s_actT = dot(mol_wn_ref[...], activT)

    for _t in range(2):
        act_molT = jnp.maximum(mol_fT, 0.0)
        s_molT = dot(mol_wa_ref[...], act_molT)
        maw_cols = []
        for m in range(M):
            sl = slice(m * L, (m + 1) * L)
            mask_m = maskT_ref[m]
            msk = jnp.where(mask_m == 0, -9e8, 0.0).astype(f32)
            score = _leaky(s_molT[:, m:m + 1] + s_actT[:, sl] + mol_b_ref[...]) + msk
            mx = jnp.max(score, axis=1, keepdims=True)
            ex = jnp.exp(score - mx)
            se = jnp.sum(ex, axis=1, keepdims=True)
            maw_cols.append(ex / se)
        maw_full = jnp.concatenate(maw_cols, axis=1)
        ctxT = _elu(dot(aftT * maw_full, mcol_ref[0]))

        giT = dot(mWih_ref[...], ctxT) + mbihT_ref[...]
        ghT = dot(mWhh_ref[...], mol_fT) + mbhhT_ref[...]
        rr = jax.nn.sigmoid(giT[0:F] + ghT[0:F])
        zz = jax.nn.sigmoid(giT[F:2 * F] + ghT[F:2 * F])
        nn = jnp.tanh(giT[2 * F:3 * F] + rr * ghT[2 * F:3 * F])
        mol_fT = (1.0 - zz) * nn + zz * mol_fT

    out_ref[...] = (dot(dnnWT_ref[...], mol_fT) + dnnbT_ref[...])[None]


def kernel(atom_list, bond_list, atom_degree_list, bond_degree_list, atom_mask,
           atom_fc_W, atom_fc_b, nb_fc_W, nb_fc_b, align_W, align_b,
           attend_W, attend_b, gru_Wih, gru_Whh, gru_bih, gru_bhh,
           mol_align_W, mol_align_b, mol_attend_W, mol_attend_b,
           mol_gru_Wih, mol_gru_Whh, mol_gru_bih, mol_gru_bhh, dnn_W, dnn_b):
    f32 = jnp.float32
    B, L, IN_ATOM = atom_list.shape
    IN_BOND = bond_list.shape[-1]
    D = atom_degree_list.shape[-1]
    F = atom_fc_W.shape[1]
    R = align_W.shape[0]
    M = 16
    NO = dnn_W.shape[1]
    NP = 128

    atomT = atom_list.transpose(0, 2, 1)
    bondT = bond_list.transpose(0, 2, 1)
    aidx = atom_degree_list.astype(jnp.int32).transpose(0, 2, 1).reshape(B, 1, D * L)
    bidx = bond_degree_list.astype(jnp.int32).transpose(0, 2, 1).reshape(B, 1, D * L)
    maskT = atom_mask.astype(f32).reshape(B, 1, L)
    eye = jnp.eye(M, dtype=f32)
    mcol = (atom_mask.astype(f32).reshape(B // M, M, L)[:, :, :, None]
            * eye[:, None, :]).reshape(B // M, M * L, M)

    atom_fcT = atom_fc_W.T
    atom_fcbT = atom_fc_b.reshape(F, 1)
    WnbaT = nb_fc_W[:IN_ATOM].T
    WnbbT = nb_fc_W[IN_ATOM:].T
    nbfbT = nb_fc_b.reshape(F, 1)
    align_wa = align_W[:, :F, 0].reshape(R, 1, F)
    align_wn = align_W[:, F:, 0].reshape(R, 1, F)
    align_b3 = align_b.reshape(R, 1, 1)
    attend_WT = jnp.transpose(attend_W, (0, 2, 1))
    attend_bT = attend_b.reshape(R, F, 1)
    bihT = gru_bih.reshape(R, 3 * F, 1)
    bhhT = gru_bhh.reshape(R, 3 * F, 1)
    mol_wa = mol_align_W[:F, 0].reshape(1, F)
    mol_wn = mol_align_W[F:, 0].reshape(1, F)
    mol_b2 = mol_align_b.reshape(1, 1)
    mol_attWT = mol_attend_W.T
    mol_attbT = mol_attend_b.reshape(F, 1)
    mbihT = mol_gru_bih.reshape(3 * F, 1)
    mbhhT = mol_gru_bhh.reshape(3 * F, 1)
    dnnWT_p = jnp.zeros((NP, F), f32).at[:NO, :].set(dnn_W.T)
    dnnbT_p = jnp.zeros((NP, 1), f32).at[:NO, 0].set(dnn_b)

    def fixed(a):
        nd = a.ndim
        return pl.BlockSpec(a.shape, lambda i, _nd=nd: (0,) * _nd)

    weights = [atom_fcT, atom_fcbT, WnbaT, WnbbT, nbfbT,
               align_wa, align_wn, align_b3, attend_WT, attend_bT,
               gru_Wih, gru_Whh, bihT, bhhT,
               mol_wa, mol_wn, mol_b2, mol_attWT, mol_attbT,
               mol_gru_Wih, mol_gru_Whh, mbihT, mbhhT, dnnWT_p, dnnbT_p]

    out = pl.pallas_call(
        functools.partial(_body, M, L, D, F),
        grid=(B // M,),
        in_specs=[
            pl.BlockSpec((M, IN_ATOM, L), lambda i: (i, 0, 0)),
            pl.BlockSpec((M, IN_BOND, L), lambda i: (i, 0, 0)),
            pl.BlockSpec((M, 1, D * L), lambda i: (i, 0, 0)),
            pl.BlockSpec((M, 1, D * L), lambda i: (i, 0, 0)),
            pl.BlockSpec((M, 1, L), lambda i: (i, 0, 0)),
            pl.BlockSpec((1, M * L, M), lambda i: (i, 0, 0)),
        ] + [fixed(w) for w in weights],
        out_specs=pl.BlockSpec((1, NP, M), lambda i: (i, 0, 0)),
        out_shape=jax.ShapeDtypeStruct((B // M, NP, M), f32),
    )(atomT, bondT, aidx, bidx, maskT, mcol, *weights)

    return out.transpose(0, 2, 1).reshape(B, NP)[:, :NO]


# speedup vs baseline: 37.7562x; 1.3069x over previous
import functools

import jax
import jax.numpy as jnp
from jax.experimental import pallas as pl


def _leaky(x):
    return jnp.where(x >= 0, x, 0.01 * x)


def _elu(x):
    return jnp.where(x > 0, x, jnp.exp(jnp.minimum(x, 0.0)) - 1.0)


def _body(M, L, D, F,
          atomT_ref, bondT_ref, aidx_ref, bidx_ref, maskT_ref, mcol_ref,
          atom_fcT_ref, atom_fcbT_ref, WnbaT_ref, WnbbT_ref, nbfbT_ref,
          align_wa_ref, align_wn_ref, align_b_ref,
          attend_WT_ref, attend_bT_ref,
          Wih_ref, Whh_ref, bihT_ref, bhhT_ref,
          mol_wa_ref, mol_wn_ref, mol_b_ref,
          mol_attWT_ref, mol_attbT_ref,
          mWih_ref, mWhh_ref, mbihT_ref, mbhhT_ref,
          dnnWT_ref, dnnbT_ref,
          out_ref):
    f32 = jnp.float32
    ML = M * L
    DL = D * L

    dot = functools.partial(jnp.dot, preferred_element_type=f32)

    atomT = jnp.concatenate([atomT_ref[m] for m in range(M)], axis=1)
    bondT = jnp.concatenate([bondT_ref[m] for m in range(M)], axis=1)

    afT = _leaky(dot(atom_fcT_ref[...], atomT) + atom_fcbT_ref[...])
    paT = dot(WnbaT_ref[...], atomT)
    pbT = dot(WnbbT_ref[...], bondT)

    iota = jax.lax.broadcasted_iota(jnp.int32, (L, DL), 0)

    OaT = []
    nbfT = []
    amask = []
    smask = []
    for m in range(M):
        aidx_m = aidx_ref[m]
        bidx_m = bidx_ref[m]
        oaT = (iota == aidx_m).astype(f32)
        obT = (iota == bidx_m).astype(f32)
        sl = slice(m * L, (m + 1) * L)
        pabT = jnp.concatenate([paT[:, sl], pbT[:, sl]], axis=1)
        oabT = jnp.concatenate([oaT, obT], axis=0)
        nbfT.append(_leaky(dot(pabT, oabT) + nbfbT_ref[...]))
        OaT.append(oaT)
        is_pad = aidx_m == (L - 1)
        amask.append(jnp.where(is_pad, 0.0, 1.0).astype(f32))
        smask.append(jnp.where(is_pad, -9e8, 0.0).astype(f32))

    hT = afT
    atom_featT = afT
    for r in range(3):
        wa = align_wa_ref[r]
        wn = align_wn_ref[r]
        ab = align_b_ref[r]
        saT = dot(wa, atom_featT)
        if r > 0:
            activT = jnp.maximum(hT, 0.0)
            atT = dot(attend_WT_ref[r], activT) + attend_bT_ref[r]
            snvT = dot(wn, activT)
        ctx_cols = []
        for m in range(M):
            sl = slice(m * L, (m + 1) * L)
            if r == 0:
                nbf_m = nbfT[m]
                sn_row = dot(wn, nbf_m)
                nft_m = dot(attend_WT_ref[r], nbf_m) + attend_bT_ref[r]
            else:
                sn_row = dot(snvT[:, sl], OaT[m])
            sa_m = saT[:, sl]
            score = [
                _leaky(sa_m + sn_row[:, d * L:(d + 1) * L] + ab)
                + smask[m][:, d * L:(d + 1) * L]
                for d in range(D)
            ]
            mx01 = jnp.maximum(score[0], score[1])
            mx23 = jnp.maximum(score[2], score[3])
            mx45 = jnp.maximum(score[4], score[5])
            mx = jnp.maximum(jnp.maximum(mx01, mx23), mx45)
            ex = [jnp.exp(score[d] - mx) for d in range(D)]
            se = (ex[0] + ex[1]) + (ex[2] + ex[3]) + (ex[4] + ex[5])
            inv = 1.0 / se
            aw = [ex[d] * inv * amask[m][:, d * L:(d + 1) * L] for d in range(D)]
            if r == 0:
                ctx_m = aw[0] * nft_m[:, 0 * L:1 * L]
                for d in range(1, D):
                    ctx_m = ctx_m + aw[d] * nft_m[:, d * L:(d + 1) * L]
            else:
                WgT = aw[0] * OaT[m][:, 0 * L:1 * L]
                for d in range(1, D):
                    WgT = WgT + aw[d] * OaT[m][:, d * L:(d + 1) * L]
                ctx_m = dot(atT[:, sl], WgT)
            ctx_cols.append(ctx_m)
        ctxT = _elu(jnp.concatenate(ctx_cols, axis=1))

        giT = dot(Wih_ref[r], ctxT) + bihT_ref[r]
        ghT = dot(Whh_ref[r], hT) + bhhT_ref[r]
        rr = jax.nn.sigmoid(giT[0:F] + ghT[0:F])
        zz = jax.nn.sigmoid(giT[F:2 * F] + ghT[F:2 * F])
        nn = jnp.tanh(giT[2 * F:3 * F] + rr * ghT[2 * F:3 * F])
        hT = (1.0 - zz) * nn + zz * hT
        atom_featT = hT

    activT = jnp.maximum(hT, 0.0)

    mol_fT = dot(activT, mcol_ref[0])
    aftT = dot(mol_attWT_ref[...], activT) + mol_attbT_ref[...]
    s_actT = dot(mol_wn_ref[...], activT)

    for _t in range(2):
        act_molT = jnp.maximum(mol_fT, 0.0)
        s_molT = dot(mol_wa_ref[...], act_molT)
        maw_cols = []
        for m in range(M):
            sl = slice(m * L, (m + 1) * L)
            mask_m = maskT_ref[m]
            msk = jnp.where(mask_m == 0, -9e8, 0.0).astype(f32)
            score = _leaky(s_molT[:, m:m + 1] + s_actT[:, sl] + mol_b_ref[...]) + msk
            mx = jnp.max(score, axis=1, keepdims=True)
            ex = jnp.exp(score - mx)
            se = jnp.sum(ex, axis=1, keepdims=True)
            maw_cols.append(ex / se)
        maw_full = jnp.concatenate(maw_cols, axis=1)
        ctxT = _elu(dot(aftT * maw_full, mcol_ref[0]))

        giT = dot(mWih_ref[...], ctxT) + mbihT_ref[...]
        ghT = dot(mWhh_ref[...], mol_fT) + mbhhT_ref[...]
        rr = jax.nn.sigmoid(giT[0:F] + ghT[0:F])
        zz = jax.nn.sigmoid(giT[F:2 * F] + ghT[F:2 * F])
        nn = jnp.tanh(giT[2 * F:3 * F] + rr * ghT[2 * F:3 * F])
        mol_fT = (1.0 - zz) * nn + zz * mol_fT

    out_ref[...] = (dot(dnnWT_ref[...], mol_fT) + dnnbT_ref[...])[None]


def kernel(atom_list, bond_list, atom_degree_list, bond_degree_list, atom_mask,
           atom_fc_W, atom_fc_b, nb_fc_W, nb_fc_b, align_W, align_b,
           attend_W, attend_b, gru_Wih, gru_Whh, gru_bih, gru_bhh,
           mol_align_W, mol_align_b, mol_attend_W, mol_attend_b,
           mol_gru_Wih, mol_gru_Whh, mol_gru_bih, mol_gru_bhh, dnn_W, dnn_b):
    f32 = jnp.float32
    B, L, IN_ATOM = atom_list.shape
    IN_BOND = bond_list.shape[-1]
    D = atom_degree_list.shape[-1]
    F = atom_fc_W.shape[1]
    R = align_W.shape[0]
    M = 32
    NO = dnn_W.shape[1]
    NP = 128

    atomT = atom_list.transpose(0, 2, 1)
    bondT = bond_list.transpose(0, 2, 1)
    aidx = atom_degree_list.astype(jnp.int32).transpose(0, 2, 1).reshape(B, 1, D * L)
    bidx = bond_degree_list.astype(jnp.int32).transpose(0, 2, 1).reshape(B, 1, D * L)
    maskT = atom_mask.astype(f32).reshape(B, 1, L)
    eye = jnp.eye(M, dtype=f32)
    mcol = (atom_mask.astype(f32).reshape(B // M, M, L)[:, :, :, None]
            * eye[:, None, :]).reshape(B // M, M * L, M)

    atom_fcT = atom_fc_W.T
    atom_fcbT = atom_fc_b.reshape(F, 1)
    WnbaT = nb_fc_W[:IN_ATOM].T
    WnbbT = nb_fc_W[IN_ATOM:].T
    nbfbT = nb_fc_b.reshape(F, 1)
    align_wa = align_W[:, :F, 0].reshape(R, 1, F)
    align_wn = align_W[:, F:, 0].reshape(R, 1, F)
    align_b3 = align_b.reshape(R, 1, 1)
    attend_WT = jnp.transpose(attend_W, (0, 2, 1))
    attend_bT = attend_b.reshape(R, F, 1)
    bihT = gru_bih.reshape(R, 3 * F, 1)
    bhhT = gru_bhh.reshape(R, 3 * F, 1)
    mol_wa = mol_align_W[:F, 0].reshape(1, F)
    mol_wn = mol_align_W[F:, 0].reshape(1, F)
    mol_b2 = mol_align_b.reshape(1, 1)
    mol_attWT = mol_attend_W.T
    mol_attbT = mol_attend_b.reshape(F, 1)
    mbihT = mol_gru_bih.reshape(3 * F, 1)
    mbhhT = mol_gru_bhh.reshape(3 * F, 1)
    dnnWT_p = jnp.zeros((NP, F), f32).at[:NO, :].set(dnn_W.T)
    dnnbT_p = jnp.zeros((NP, 1), f32).at[:NO, 0].set(dnn_b)

    def fixed(a):
        nd = a.ndim
        return pl.BlockSpec(a.shape, lambda i, _nd=nd: (0,) * _nd)

    weights = [atom_fcT, atom_fcbT, WnbaT, WnbbT, nbfbT,
               align_wa, align_wn, align_b3, attend_WT, attend_bT,
               gru_Wih, gru_Whh, bihT, bhhT,
               mol_wa, mol_wn, mol_b2, mol_attWT, mol_attbT,
               mol_gru_Wih, mol_gru_Whh, mbihT, mbhhT, dnnWT_p, dnnbT_p]

    out = pl.pallas_call(
        functools.partial(_body, M, L, D, F),
        grid=(B // M,),
        in_specs=[
            pl.BlockSpec((M, IN_ATOM, L), lambda i: (i, 0, 0)),
            pl.BlockSpec((M, IN_BOND, L), lambda i: (i, 0, 0)),
            pl.BlockSpec((M, 1, D * L), lambda i: (i, 0, 0)),
            pl.BlockSpec((M, 1, D * L), lambda i: (i, 0, 0)),
            pl.BlockSpec((M, 1, L), lambda i: (i, 0, 0)),
            pl.BlockSpec((1, M * L, M), lambda i: (i, 0, 0)),
        ] + [fixed(w) for w in weights],
        out_specs=pl.BlockSpec((1, NP, M), lambda i: (i, 0, 0)),
        out_shape=jax.ShapeDtypeStruct((B // M, NP, M), f32),
    )(atomT, bondT, aidx, bidx, maskT, mcol, *weights)

    return out.transpose(0, 2, 1).reshape(B, NP)[:, :NO]
